# baseline (speedup 1.0000x reference)
import functools

import jax
import jax.numpy as jnp
from jax.experimental import pallas as pl
from jax.experimental.pallas import tpu as pltpu
from jax.experimental.pallas import tpu_sc as plsc


def _stitch_body(n_rows, n_cols, m0, part_hbm, idx0_hbm, data_hbm, out_hbm,
                 part_s, idx0_s, data_s, out_s, nz_s, sem):
    copies = [pltpu.async_copy(part_hbm, part_s, sem),
              pltpu.async_copy(idx0_hbm, idx0_s, sem),
              pltpu.async_copy(data_hbm, data_s, sem)]
    for c in copies:
        c.wait()

    for i in range(m0):
        nz_s[i] = 0
    cnt = jnp.int32(0)
    for i in range(n_rows):
        hit = part_s[i] == 0

        @pl.when(hit & (cnt < m0))
        def _(cnt=cnt, i=i):
            nz_s[cnt] = i

        cnt = cnt + jnp.where(hit, 1, 0)

    for r in range(n_rows):
        for j in range(n_cols):
            out_s[r, j] = 0.0
    for i in range(m0):
        r = nz_s[i]
        d = idx0_s[i]

        @pl.when((d >= 0) & (d < n_rows))
        def _(r=r, d=d):
            for j in range(n_cols):
                out_s[d, j] = data_s[r, j]

    pltpu.sync_copy(out_s, out_hbm)


def kernel(data, partitions, index0, index1):
    n_rows, n_cols = data.shape
    m0 = index0.shape[0]
    assert index1.shape[0] == 0

    body = functools.partial(_stitch_body, n_rows, n_cols, m0)
    return pl.kernel(
        body,
        out_type=jax.ShapeDtypeStruct((n_rows, n_cols), jnp.float32),
        mesh=plsc.ScalarSubcoreMesh(axis_name="c", num_cores=1),
        scratch_types=[
            pltpu.SMEM((n_rows,), jnp.int32),
            pltpu.SMEM((m0,), jnp.int32),
            pltpu.SMEM((n_rows, n_cols), jnp.float32),
            pltpu.SMEM((n_rows, n_cols), jnp.float32),
            pltpu.SMEM((m0,), jnp.int32),
            pltpu.SemaphoreType.DMA,
        ],
        compiler_params=pltpu.CompilerParams(needs_layout_passes=False),
    )(partitions, index0, data)
